# Initial kernel scaffold; baseline (speedup 1.0000x reference)
import jax
import jax.numpy as jnp
from jax.experimental import pallas as pl


def kernel(user_ids, item_ids, U, M, A, B, W1, b1, W2, b2):
    raise NotImplementedError("write your pallas kernel here")


# speedup vs baseline: 1.1827x; 1.1827x over previous
import functools

import jax
import jax.numpy as jnp
from jax import lax
from jax.experimental import pallas as pl
from jax.experimental.pallas import tpu as pltpu
from jax.experimental.pallas import tpu_sc as plsc

_BATCH = 16384
_D = 128
_H1 = 384
_H2 = 256

_NC = 2
_NS = 16
_NW = _NC * _NS
_BPW = _BATCH // _NW
_CH = 128
_NCH = _BPW // _CH

_BB = 2048
_GRID = _BATCH // _BB


def _sc_gather_body(u_hbm, m_hbm, a_hbm, b_hbm, uidx_hbm, iidx_hbm,
                    users_hbm, items_hbm, ag_hbm, bg_hbm,
                    idx_u, idx_i, buf_u, buf_m, buf_a, buf_b,
                    sem_u, sem_m, sem_a, sem_b):
    wid = lax.axis_index("s") * _NC + lax.axis_index("c")
    pltpu.sync_copy(uidx_hbm.at[wid], idx_u)
    pltpu.sync_copy(iidx_hbm.at[wid], idx_i)
    for c in range(_NCH):
        cu = pltpu.async_copy(u_hbm.at[idx_u.at[c]], buf_u, sem_u)
        cm = pltpu.async_copy(m_hbm.at[idx_i.at[c]], buf_m, sem_m)
        ca = pltpu.async_copy(a_hbm.at[idx_u.at[c]], buf_a.at[c], sem_a)
        cb = pltpu.async_copy(b_hbm.at[idx_i.at[c]], buf_b.at[c], sem_b)
        cu.wait()
        cm.wait()
        ca.wait()
        cb.wait()
        row0 = wid * _BPW + c * _CH
        pltpu.sync_copy(buf_u, users_hbm.at[pl.ds(row0, _CH)])
        pltpu.sync_copy(buf_m, items_hbm.at[pl.ds(row0, _CH)])
    pltpu.sync_copy(buf_a, ag_hbm.at[wid])
    pltpu.sync_copy(buf_b, bg_hbm.at[wid])


_sc_gather = pl.kernel(
    _sc_gather_body,
    out_type=(
        jax.ShapeDtypeStruct((_BATCH, _D), jnp.float32),
        jax.ShapeDtypeStruct((_BATCH, _D), jnp.float32),
        jax.ShapeDtypeStruct((_NW, _NCH, _CH), jnp.float32),
        jax.ShapeDtypeStruct((_NW, _NCH, _CH), jnp.float32),
    ),
    mesh=plsc.VectorSubcoreMesh(core_axis_name="c", subcore_axis_name="s"),
    scratch_types=[
        pltpu.VMEM((_NCH, _CH), jnp.int32),
        pltpu.VMEM((_NCH, _CH), jnp.int32),
        pltpu.VMEM((_CH, _D), jnp.float32),
        pltpu.VMEM((_CH, _D), jnp.float32),
        pltpu.VMEM((_NCH, _CH), jnp.float32),
        pltpu.VMEM((_NCH, _CH), jnp.float32),
        pltpu.SemaphoreType.DMA,
        pltpu.SemaphoreType.DMA,
        pltpu.SemaphoreType.DMA,
        pltpu.SemaphoreType.DMA,
    ],
)


def _tc_mlp_body(u_ref, it_ref, ag_ref, bg_ref, w1a_ref, w1b_ref, w1c_ref,
                 b1_ref, w2_ref, b2_ref, pred_ref, score_ref):
    u = u_ref[...]
    it = it_ref[...]
    ui = u * it
    pred_ref[...] = jnp.sum(ui, axis=1) + ag_ref[...] + bg_ref[...]
    h = jnp.dot(u, w1a_ref[...], preferred_element_type=jnp.float32)
    h = h + jnp.dot(it, w1b_ref[...], preferred_element_type=jnp.float32)
    h = h + jnp.dot(ui, w1c_ref[...], preferred_element_type=jnp.float32)
    h = jnp.maximum(h + b1_ref[...], 0.0)
    score_ref[...] = jnp.sum(h * w2_ref[...], axis=1) + b2_ref[0]


_tc_mlp = pl.pallas_call(
    _tc_mlp_body,
    grid=(_GRID,),
    in_specs=[
        pl.BlockSpec((_BB, _D), lambda i: (i, 0)),
        pl.BlockSpec((_BB, _D), lambda i: (i, 0)),
        pl.BlockSpec((_BB,), lambda i: (i,)),
        pl.BlockSpec((_BB,), lambda i: (i,)),
        pl.BlockSpec((_D, _H2), lambda i: (0, 0)),
        pl.BlockSpec((_D, _H2), lambda i: (0, 0)),
        pl.BlockSpec((_D, _H2), lambda i: (0, 0)),
        pl.BlockSpec((1, _H2), lambda i: (0, 0)),
        pl.BlockSpec((1, _H2), lambda i: (0, 0)),
        pl.BlockSpec(memory_space=pltpu.SMEM),
    ],
    out_specs=[
        pl.BlockSpec((_BB,), lambda i: (i,)),
        pl.BlockSpec((_BB,), lambda i: (i,)),
    ],
    out_shape=[
        jax.ShapeDtypeStruct((_BATCH,), jnp.float32),
        jax.ShapeDtypeStruct((_BATCH,), jnp.float32),
    ],
)


@jax.jit
def kernel(user_ids, item_ids, U, M, A, B, W1, b1, W2, b2):
    uidx3 = user_ids.astype(jnp.int32).reshape(_NW, _NCH, _CH)
    iidx3 = item_ids.astype(jnp.int32).reshape(_NW, _NCH, _CH)
    a1 = A.reshape(-1)
    b1d = B.reshape(-1)
    users, items, ag, bg = _sc_gather(U, M, a1, b1d, uidx3, iidx3)
    ag = ag.reshape(_BATCH)
    bg = bg.reshape(_BATCH)
    w1a = W1[:_D]
    w1b = W1[_D:2 * _D]
    w1c = W1[2 * _D:]
    pred, score = _tc_mlp(users, items, ag, bg, w1a, w1b, w1c,
                          b1.reshape(1, _H2), W2.reshape(1, _H2), b2)
    return pred, score
